# baseline (device time: 342822 ns/iter reference)
import jax
import jax.numpy as jnp
import numpy as np
from jax import lax
from jax.experimental import pallas as pl
from jax.experimental.pallas import tpu as pltpu

N_DEV = 4
NCH = 8
N_HOPS = 1


def kernel(x, A, B, C):
    Bb, S, D = x.shape
    N = A.shape[1]
    LC = S // NCH
    BM = Bb * NCH
    At = A.T

    xm = x.reshape(BM, LC, D)
    bm = B.reshape(BM, LC, N)
    cm = C.reshape(BM, LC, N)

    def body(x_ref, at_ref, b_ref, c_ref, y_ref, comm_ref, send_sem, recv_sem):
        my = lax.axis_index("i")
        left = lax.rem(my + (N_DEV - 1), N_DEV)
        right = lax.rem(my + 1, N_DEV)

        dA = jnp.exp(at_ref[...])
        dAc = jnp.exp(at_ref[...] * np.float32(LC))

        def stepA(t, h):
            x_t = x_ref[:, pl.ds(t, 1), :]
            bT = jnp.swapaxes(b_ref[:, pl.ds(t, 1), :], 1, 2)
            return h * dA[None] + bT * x_t

        hA = lax.fori_loop(0, LC, stepA,
                           jnp.zeros((BM, N, D), jnp.float32), unroll=8)
        hA4 = hA.reshape(Bb, NCH, N, D)

        lo = [jnp.zeros((Bb, N, D), jnp.float32)]
        for c in range(1, NCH):
            lo.append(hA4[:, c - 1] + dAc[None] * lo[c - 1])
        h_send = hA4[:, NCH - 1] + dAc[None] * lo[NCH - 1]

        comm_ref[0] = h_send

        barrier = pltpu.get_barrier_semaphore()
        for nbr in (left, right):
            pl.semaphore_signal(barrier, inc=1, device_id=(nbr,),
                                device_id_type=pl.DeviceIdType.MESH)
        pl.semaphore_wait(barrier, 2)

        rdma = pltpu.make_async_remote_copy(
            src_ref=comm_ref.at[0],
            dst_ref=comm_ref.at[1],
            send_sem=send_sem,
            recv_sem=recv_sem,
            device_id=(right,),
            device_id_type=pl.DeviceIdType.MESH)
        rdma.start()
        rdma.wait()

        m = jnp.where(my >= 1, np.float32(1), np.float32(0))
        hi = [m * comm_ref[1]] + lo[1:]
        h0 = jnp.stack(hi, axis=1).reshape(BM, N, D)

        def stepB(t, h):
            x_t = x_ref[:, pl.ds(t, 1), :]
            bT = jnp.swapaxes(b_ref[:, pl.ds(t, 1), :], 1, 2)
            cT = jnp.swapaxes(c_ref[:, pl.ds(t, 1), :], 1, 2)
            h = h * dA[None] + bT * x_t
            y_t = jnp.sum(h * cT, axis=1, keepdims=True)
            y_ref[:, pl.ds(t, 1), :] = y_t
            return h

        lax.fori_loop(0, LC, stepB, h0, unroll=8)

    out = pl.pallas_call(
        body,
        out_shape=jax.ShapeDtypeStruct((BM, LC, D), jnp.float32),
        in_specs=[pl.BlockSpec(memory_space=pltpu.VMEM)] * 4,
        out_specs=pl.BlockSpec(memory_space=pltpu.VMEM),
        scratch_shapes=[
            pltpu.VMEM((2, Bb, N, D), jnp.float32),
            pltpu.SemaphoreType.DMA,
            pltpu.SemaphoreType.DMA,
        ],
        compiler_params=pltpu.CompilerParams(
            collective_id=0, vmem_limit_bytes=100 * 1024 * 1024),
    )(xm, At, bm, cm)
    return out.reshape(Bb, S, D)
